# baseline (speedup 1.0000x reference)
import functools

import jax
import jax.numpy as jnp
from jax import lax
from jax.experimental import pallas as pl
from jax.experimental.pallas import tpu as pltpu
from jax.experimental.pallas import tpu_sc as plsc

_TEMPERATURE = 0.8
_TOP_P = 0.95
_B = 64
_V = 100000
_L = 16
_NCHUNK = _V // _L
_NB1 = 16384
_NC = 2
_NS = 16
_ROWS_PER_W = _B // (_NC * _NS)


def _iota16():
    return lax.broadcasted_iota(jnp.int32, (_L,), 0)


def _hist_walk(hist_ref, num_chunks, base_above, target):
    iota = _iota16()

    def body(j, carry):
        found, bstar, above, incl, acc = carry
        c = num_chunks - 1 - j
        hv = hist_ref[pl.ds(c * _L, _L)]
        rv = lax.rev(hv, (0,))
        cum = plsc.cumsum(rv) + acc
        crossed = cum > target
        any_crossed = jnp.any(crossed)
        lane = jnp.where(any_crossed, jnp.min(plsc.all_reduce_ffs(crossed)), 0)
        sel = iota == lane
        cum_l = jnp.sum(jnp.where(sel, cum, 0.0))
        hv_l = jnp.sum(jnp.where(sel, rv, 0.0))
        take = any_crossed & jnp.logical_not(found)
        bstar = jnp.where(take, c * _L + (_L - 1) - lane, bstar)
        above = jnp.where(take, cum_l - hv_l, above)
        incl = jnp.where(take, cum_l, incl)
        acc = acc + jnp.sum(hv)
        return found | any_crossed, bstar, above, incl, acc

    init = (jnp.bool_(False), jnp.int32(0), jnp.float32(0.0),
            jnp.float32(0.0), base_above)
    found, bstar, above, incl, acc = lax.fori_loop(0, num_chunks, body, init)
    h0 = jnp.sum(jnp.where(iota == 0, hist_ref[pl.ds(0, _L)], 0.0))
    bstar = jnp.where(found, bstar, 0)
    above = jnp.where(found, above, acc - h0)
    incl = jnp.where(found, incl, acc)
    return bstar, above, incl


def _zero(ref, n):
    def body(i, _):
        ref[pl.ds(i * _L, _L)] = jnp.zeros((_L,), jnp.float32)
        return 0
    lax.fori_loop(0, n // _L, body, 0)


def _sc_body(x_hbm, o_hbm, row_v, h1_v, h2_v, h3_v):
    wid = lax.axis_index("s") * _NC + lax.axis_index("c")
    for k in range(_ROWS_PER_W):
        r = wid * _ROWS_PER_W + k
        pltpu.sync_copy(x_hbm.at[r], row_v)

        def max_body(i, acc):
            return jnp.maximum(acc, row_v[pl.ds(i * _L, _L)])
        macc = lax.fori_loop(0, _NCHUNK, max_body,
                             jnp.full((_L,), -jnp.inf, jnp.float32))
        mx = jnp.max(macc)

        _zero(h1_v, _NB1)

        def exp_body(i, zacc):
            sl = pl.ds(i * _L, _L)
            w = jnp.exp((row_v[sl] - mx) * (1.0 / _TEMPERATURE))
            row_v[sl] = w
            idx = lax.shift_right_logical(
                lax.bitcast_convert_type(w, jnp.int32), 16)
            plsc.addupdate_scatter(h1_v, [idx], w)
            return zacc + w
        zacc = lax.fori_loop(0, _NCHUNK, exp_body,
                             jnp.zeros((_L,), jnp.float32))
        target = _TOP_P * jnp.sum(zacc)

        b1, above1, _ = _hist_walk(h1_v, _NB1 // _L, jnp.float32(0.0), target)

        _zero(h2_v, 256)

        def h2_body(i, _):
            sl = pl.ds(i * _L, _L)
            w = row_v[sl]
            b = lax.bitcast_convert_type(w, jnp.int32)
            m = lax.shift_right_logical(b, 16) == b1
            idx = lax.shift_right_logical(b, 8) & 0xFF
            plsc.addupdate_scatter(h2_v, [idx], w, mask=m)
            return 0
        lax.fori_loop(0, _NCHUNK, h2_body, 0)
        b2, above2, _ = _hist_walk(h2_v, 256 // _L, above1, target)

        _zero(h3_v, 256)
        top24 = (b1 << 8) | b2

        def h3_body(i, _):
            sl = pl.ds(i * _L, _L)
            w = row_v[sl]
            b = lax.bitcast_convert_type(w, jnp.int32)
            m = lax.shift_right_logical(b, 8) == top24
            idx = b & 0xFF
            plsc.addupdate_scatter(h3_v, [idx], w, mask=m)
            return 0
        lax.fori_loop(0, _NCHUNK, h3_body, 0)
        b3, _, incl3 = _hist_walk(h3_v, 256 // _L, above2, target)

        tau = lax.bitcast_convert_type((top24 << 8) | b3, jnp.float32)
        inv_w = 1.0 / (incl3 + jnp.zeros((_L,), jnp.float32))

        def out_body(i, _):
            sl = pl.ds(i * _L, _L)
            w = row_v[sl]
            row_v[sl] = jnp.where(w >= tau, w * inv_w, 0.0)
            return 0
        lax.fori_loop(0, _NCHUNK, out_body, 0)
        pltpu.sync_copy(row_v, o_hbm.at[r])


@jax.jit
def kernel(logits):
    mesh = plsc.VectorSubcoreMesh(core_axis_name="c", subcore_axis_name="s",
                                  num_cores=_NC, num_subcores=_NS)
    f = pl.kernel(
        _sc_body,
        out_type=jax.ShapeDtypeStruct((_B, _V), jnp.float32),
        mesh=mesh,
        compiler_params=pltpu.CompilerParams(needs_layout_passes=False),
        scratch_types=[
            pltpu.VMEM((_V,), jnp.float32),
            pltpu.VMEM((_NB1,), jnp.float32),
            pltpu.VMEM((256,), jnp.float32),
            pltpu.VMEM((256,), jnp.float32),
        ],
    )
    return f(logits)


# speedup vs baseline: 5.3346x; 5.3346x over previous
import functools

import jax
import jax.numpy as jnp
from jax import lax
from jax.experimental import pallas as pl
from jax.experimental.pallas import tpu as pltpu
from jax.experimental.pallas import tpu_sc as plsc

_TEMPERATURE = 0.8
_TOP_P = 0.95
_B = 64
_V = 100000
_L = 16
_NCHUNK = _V // _L
_NB1 = 4096
_NB23 = 512
_NC = 2
_NS = 16
_ROWS_PER_W = _B // (_NC * _NS)
_UNROLL = 4


def _iota16():
    return lax.broadcasted_iota(jnp.int32, (_L,), 0)


def _hist_walk(hist_ref, num_chunks, base_above, target):
    def cond(carry):
        j, _, done = carry
        return jnp.logical_not(done) & (j < num_chunks)

    def body(carry):
        j, acc, _ = carry
        c = num_chunks - 1 - j
        s = jnp.sum(hist_ref[pl.ds(c * _L, _L)])
        cross = (acc + s) > target
        return (jnp.where(cross, j, j + 1),
                jnp.where(cross, acc, acc + s),
                cross)

    j, acc, found = lax.while_loop(
        cond, body, (jnp.int32(0), base_above, jnp.bool_(False)))

    c = num_chunks - 1 - jnp.where(found, j, num_chunks - 1)
    hv = hist_ref[pl.ds(c * _L, _L)]
    rv = lax.rev(hv, (0,))
    cum = plsc.cumsum(rv) + acc
    crossed = cum > target
    lane = jnp.where(jnp.any(crossed),
                     jnp.min(plsc.all_reduce_ffs(crossed)), 0)
    sel = _iota16() == lane
    cum_l = jnp.sum(jnp.where(sel, cum, 0.0))
    hv_l = jnp.sum(jnp.where(sel, rv, 0.0))
    bstar = c * _L + (_L - 1) - lane
    h0 = jnp.sum(jnp.where(_iota16() == 0, hist_ref[pl.ds(0, _L)], 0.0))
    bstar = jnp.where(found, bstar, 0)
    above = jnp.where(found, cum_l - hv_l, acc - h0)
    incl = jnp.where(found, cum_l, acc)
    return bstar, above, incl


def _zero(ref, n):
    @plsc.parallel_loop(0, n // _L, unroll=_UNROLL)
    def _(i):
        ref[pl.ds(i * _L, _L)] = jnp.zeros((_L,), jnp.float32)


def _sc_body(x_hbm, o_hbm, row_v, h1_v, h2_v, h3_v):
    wid = lax.axis_index("s") * _NC + lax.axis_index("c")
    for k in range(_ROWS_PER_W):
        r = wid * _ROWS_PER_W + k
        pltpu.sync_copy(x_hbm.at[r], row_v)

        minf = jnp.full((_L,), -jnp.inf, jnp.float32)

        @plsc.parallel_loop(0, _NCHUNK, 2, unroll=_UNROLL, carry=(minf, minf))
        def macc(i, acc):
            a0 = jnp.maximum(acc[0], row_v[pl.ds(i * _L, _L)])
            a1 = jnp.maximum(acc[1], row_v[pl.ds((i + 1) * _L, _L)])
            return a0, a1
        mx = jnp.max(jnp.maximum(macc[0], macc[1]))

        _zero(h1_v, _NB1)
        zinit = jnp.zeros((_L,), jnp.float32)

        @plsc.parallel_loop(0, _NCHUNK, 2, unroll=_UNROLL, carry=(zinit, zinit))
        def zacc(i, z):
            out = []
            for t in range(2):
                sl = pl.ds((i + t) * _L, _L)
                w = jnp.exp((row_v[sl] - mx) * (1.0 / _TEMPERATURE))
                row_v[sl] = w
                idx = lax.shift_right_logical(
                    lax.bitcast_convert_type(w, jnp.int32), 18)
                plsc.addupdate_scatter(h1_v, [idx], w)
                out.append(z[t] + w)
            return tuple(out)
        target = _TOP_P * jnp.sum(zacc[0] + zacc[1])

        b1, above1, _ = _hist_walk(h1_v, _NB1 // _L, jnp.float32(0.0), target)

        _zero(h2_v, _NB23)

        @plsc.parallel_loop(0, _NCHUNK, 2, unroll=_UNROLL)
        def _(i):
            for t in range(2):
                sl = pl.ds((i + t) * _L, _L)
                w = row_v[sl]
                b = lax.bitcast_convert_type(w, jnp.int32)
                m = lax.shift_right_logical(b, 18) == b1
                idx = lax.shift_right_logical(b, 9) & 0x1FF
                plsc.addupdate_scatter(h2_v, [idx], w, mask=m)
        b2, above2, _ = _hist_walk(h2_v, _NB23 // _L, above1, target)

        _zero(h3_v, _NB23)
        top23 = (b1 << 9) | b2

        @plsc.parallel_loop(0, _NCHUNK, 2, unroll=_UNROLL)
        def _(i):
            for t in range(2):
                sl = pl.ds((i + t) * _L, _L)
                w = row_v[sl]
                b = lax.bitcast_convert_type(w, jnp.int32)
                m = lax.shift_right_logical(b, 9) == top23
                idx = b & 0x1FF
                plsc.addupdate_scatter(h3_v, [idx], w, mask=m)
        b3, _, incl3 = _hist_walk(h3_v, _NB23 // _L, above2, target)

        tau = lax.bitcast_convert_type((top23 << 9) | b3, jnp.float32)
        inv_w = 1.0 / (incl3 + jnp.zeros((_L,), jnp.float32))

        @plsc.parallel_loop(0, _NCHUNK, 2, unroll=_UNROLL)
        def _(i):
            for t in range(2):
                sl = pl.ds((i + t) * _L, _L)
                w = row_v[sl]
                row_v[sl] = jnp.where(w >= tau, w * inv_w, 0.0)
        pltpu.sync_copy(row_v, o_hbm.at[r])


@jax.jit
def kernel(logits):
    mesh = plsc.VectorSubcoreMesh(core_axis_name="c", subcore_axis_name="s",
                                  num_cores=_NC, num_subcores=_NS)
    f = pl.kernel(
        _sc_body,
        out_type=jax.ShapeDtypeStruct((_B, _V), jnp.float32),
        mesh=mesh,
        compiler_params=pltpu.CompilerParams(needs_layout_passes=False),
        scratch_types=[
            pltpu.VMEM((_V,), jnp.float32),
            pltpu.VMEM((_NB1,), jnp.float32),
            pltpu.VMEM((_NB23,), jnp.float32),
            pltpu.VMEM((_NB23,), jnp.float32),
        ],
    )
    return f(logits)
